# Initial kernel scaffold; baseline (speedup 1.0000x reference)
import jax
import jax.numpy as jnp
from jax.experimental import pallas as pl


def kernel(x, edge_index, W1, b1, W2, b2):
    raise NotImplementedError("write your pallas kernel here")


# speedup vs baseline: 17.9908x; 17.9908x over previous
import functools

import jax
import jax.numpy as jnp
from jax import lax
from jax.experimental import pallas as pl
from jax.experimental.pallas import tpu as pltpu
from jax.experimental.pallas import tpu_sc as plsc

N = 10000
E = 320000
F = 128
HID = 128
CLS = 40
NA = 10240
NC, NS = 2, 16
NW = NC * NS
CH = 80
KCH = E // NW // CH
RPT = NA // NS
NPW = NA // NW

_MESH = plsc.VectorSubcoreMesh(core_axis_name="c", subcore_axis_name="s")


@functools.partial(
    pl.kernel,
    out_type=jax.ShapeDtypeStruct((NA,), jnp.float32),
    mesh=_MESH,
    scratch_types=[
        pltpu.VMEM((KCH, 1, CH), jnp.int32),
        pltpu.VMEM((CH,), jnp.float32),
        pltpu.VMEM((NPW,), jnp.float32),
        pltpu.VMEM_SHARED((NA,), jnp.float32),
    ],
)
def _deg_dinv(dst4, zeros_na, dinv_out, didx, ones_v, dbuf, hist):
    c = lax.axis_index("c")
    s = lax.axis_index("s")
    wid = c * NS + s
    npt = NA // NS
    pltpu.sync_copy(zeros_na.at[pl.ds(s * npt, npt)], hist.at[pl.ds(s * npt, npt)])
    for j in range(CH // 16):
        ones_v[pl.ds(j * 16, 16)] = jnp.full((16,), 1.0, jnp.float32)
    plsc.subcore_barrier()

    for w in range(2):
        pltpu.sync_copy(dst4.at[s * 2 + w], didx)

        def body(k, carry):
            pltpu.sync_copy(ones_v, hist.at[didx.at[k, 0]], add=True)
            return carry

        lax.fori_loop(0, KCH, body, 0)
    plsc.subcore_barrier()

    pltpu.sync_copy(hist.at[pl.ds(wid * NPW, NPW)], dbuf)
    for i in range(NPW // 16):
        d = dbuf[pl.ds(i * 16, 16)] + 1.0
        seed = jnp.full((16,), 0x5F3759DF, jnp.int32) - lax.shift_right_logical(
            lax.bitcast_convert_type(d, jnp.int32), 1)
        y = lax.bitcast_convert_type(seed, jnp.float32)
        for _ in range(3):
            y = y * (1.5 - 0.5 * d * y * y)
        dbuf[pl.ds(i * 16, 16)] = y
    pltpu.sync_copy(dbuf, dinv_out.at[pl.ds(wid * NPW, NPW)])


def _make_agg(width):
    @functools.partial(
        pl.kernel,
        out_type=jax.ShapeDtypeStruct((NC, NA, width), jnp.float32),
        mesh=_MESH,
        scratch_types=[
            pltpu.VMEM((KCH, 1, CH), jnp.int32),
            pltpu.VMEM((KCH, 1, CH), jnp.int32),
            pltpu.VMEM((CH, width), jnp.float32),
            pltpu.VMEM_SHARED((NA, width), jnp.float32),
            pltpu.SemaphoreType.DMA,
        ],
    )
    def agg(tbl, src4, dst4, zeros_nw, out, sidx, didx, rows, acc, sem):
        c = lax.axis_index("c")
        s = lax.axis_index("s")
        wid = c * NS + s
        pltpu.sync_copy(zeros_nw.at[pl.ds(s * RPT, RPT)], acc.at[pl.ds(s * RPT, RPT)])
        pltpu.sync_copy(src4.at[wid], sidx)
        pltpu.sync_copy(dst4.at[wid], didx)
        plsc.subcore_barrier()

        def body(k, carry):
            pltpu.async_copy(tbl.at[sidx.at[k, 0]], rows, sem).wait()
            pltpu.sync_copy(rows, acc.at[didx.at[k, 0]], add=True)
            return carry

        lax.fori_loop(0, KCH, body, 0)
        plsc.subcore_barrier()
        pltpu.sync_copy(acc.at[pl.ds(s * RPT, RPT)], out.at[c, pl.ds(s * RPT, RPT)])

    return agg


_agg128 = _make_agg(HID)

RB = 400
GRID = N // RB


def _mm_scale(x_ref, w_ref, dinv_ref, o_ref):
    o_ref[...] = jnp.dot(x_ref[...], w_ref[...],
                         preferred_element_type=jnp.float32) * dinv_ref[...]


def _layer2(p_ref, h1s_ref, dinv_ref, b1_ref, o_ref):
    a = (p_ref[0] + p_ref[1] + h1s_ref[...]) * dinv_ref[...] + b1_ref[...]
    o_ref[...] = jnp.maximum(a, 0.0) * dinv_ref[...]


def _combine(q_ref, g_ref, dinv_ref, w2_ref, b2_ref, o_ref):
    a = (q_ref[0] + q_ref[1] + g_ref[...]) * dinv_ref[...]
    o_ref[...] = jnp.dot(a, w2_ref[...],
                         preferred_element_type=jnp.float32) + b2_ref[...]


def kernel(x, edge_index, W1, b1, W2, b2):
    ei = edge_index.astype(jnp.int32)
    src4 = ei[0].reshape(NW, KCH, 1, CH)
    dst4 = ei[1].reshape(NW, KCH, 1, CH)
    zeros_na = jnp.zeros((NA,), jnp.float32)
    zeros_h = jnp.zeros((NA, HID), jnp.float32)
    b1_2d = b1.reshape(1, HID)
    b2_2d = b2.reshape(1, CLS)

    dinv_col = _deg_dinv(dst4, zeros_na).reshape(NA, 1)

    h1s = pl.pallas_call(
        _mm_scale,
        grid=(GRID,),
        in_specs=[
            pl.BlockSpec((RB, F), lambda r: (r, 0)),
            pl.BlockSpec((F, HID), lambda r: (0, 0)),
            pl.BlockSpec((RB, 1), lambda r: (r, 0)),
        ],
        out_specs=pl.BlockSpec((RB, HID), lambda r: (r, 0)),
        out_shape=jax.ShapeDtypeStruct((N, HID), jnp.float32),
    )(x, W1, dinv_col)

    p = _agg128(h1s, src4, dst4, zeros_h)

    g = pl.pallas_call(
        _layer2,
        grid=(GRID,),
        in_specs=[
            pl.BlockSpec((NC, RB, HID), lambda r: (0, r, 0)),
            pl.BlockSpec((RB, HID), lambda r: (r, 0)),
            pl.BlockSpec((RB, 1), lambda r: (r, 0)),
            pl.BlockSpec((1, HID), lambda r: (0, 0)),
        ],
        out_specs=pl.BlockSpec((RB, HID), lambda r: (r, 0)),
        out_shape=jax.ShapeDtypeStruct((N, HID), jnp.float32),
    )(p, h1s, dinv_col, b1_2d)

    q = _agg128(g, src4, dst4, zeros_h)

    out = pl.pallas_call(
        _combine,
        grid=(GRID,),
        in_specs=[
            pl.BlockSpec((NC, RB, HID), lambda r: (0, r, 0)),
            pl.BlockSpec((RB, HID), lambda r: (r, 0)),
            pl.BlockSpec((RB, 1), lambda r: (r, 0)),
            pl.BlockSpec((HID, CLS), lambda r: (0, 0)),
            pl.BlockSpec((1, CLS), lambda r: (0, 0)),
        ],
        out_specs=pl.BlockSpec((RB, CLS), lambda r: (r, 0)),
        out_shape=jax.ShapeDtypeStruct((N, CLS), jnp.float32),
    )(q, g, dinv_col, W2, b2_2d)
    return out


# speedup vs baseline: 26.4998x; 1.4730x over previous
import functools

import jax
import jax.numpy as jnp
from jax import lax
from jax.experimental import pallas as pl
from jax.experimental.pallas import tpu as pltpu
from jax.experimental.pallas import tpu_sc as plsc

N = 10000
E = 320000
F = 128
HID = 128
CLS = 40
NP = 10240
NAC = 10112
NC, NS = 2, 16
NW = NC * NS
CH = 80
KCH = E // NW // CH
RPT = NAC // NS
NPW = NP // NW

_MESH = plsc.VectorSubcoreMesh(core_axis_name="c", subcore_axis_name="s")


@functools.partial(
    pl.kernel,
    out_type=jax.ShapeDtypeStruct((NP,), jnp.float32),
    mesh=_MESH,
    scratch_types=[
        pltpu.VMEM((KCH, 1, CH), jnp.int32),
        pltpu.VMEM((128,), jnp.float32),
        pltpu.VMEM((NPW,), jnp.float32),
        pltpu.VMEM_SHARED((NP,), jnp.float32),
    ],
)
def _deg_dinv(dst4, zeros_na, dinv_out, didx, ones_v, dbuf, hist):
    c = lax.axis_index("c")
    s = lax.axis_index("s")
    wid = c * NS + s
    npt = NP // NS
    pltpu.sync_copy(zeros_na.at[pl.ds(s * npt, npt)], hist.at[pl.ds(s * npt, npt)])
    for j in range(8):
        ones_v[pl.ds(j * 16, 16)] = jnp.full((16,), 1.0, jnp.float32)
    plsc.subcore_barrier()

    for w in range(2):
        pltpu.sync_copy(dst4.at[s * 2 + w], didx)

        def body(k, carry):
            pltpu.sync_copy(ones_v.at[pl.ds(0, CH)], hist.at[didx.at[k, 0]],
                            add=True)
            return carry

        lax.fori_loop(0, KCH, body, 0)
    plsc.subcore_barrier()

    pltpu.sync_copy(hist.at[pl.ds(wid * NPW, NPW)], dbuf)
    for i in range(NPW // 16):
        d = dbuf[pl.ds(i * 16, 16)] + 1.0
        seed = jnp.full((16,), 0x5F3759DF, jnp.int32) - lax.shift_right_logical(
            lax.bitcast_convert_type(d, jnp.int32), 1)
        y = lax.bitcast_convert_type(seed, jnp.float32)
        for _ in range(3):
            y = y * (1.5 - 0.5 * d * y * y)
        dbuf[pl.ds(i * 16, 16)] = y
    pltpu.sync_copy(dbuf, dinv_out.at[pl.ds(wid * NPW, NPW)])


def _make_agg(width):
    @functools.partial(
        pl.kernel,
        out_type=jax.ShapeDtypeStruct((NC, NAC, width), jnp.float32),
        mesh=_MESH,
        scratch_types=[
            pltpu.VMEM((KCH * CH,), jnp.int32),
            pltpu.VMEM((KCH, 1, CH), jnp.int32),
            pltpu.VMEM((CH, width), jnp.float32),
            pltpu.VMEM((CH, width), jnp.float32),
            pltpu.VMEM_SHARED((NAC, width), jnp.float32),
            pltpu.SemaphoreType.DMA,
            pltpu.SemaphoreType.DMA,
        ],
    )
    def agg(tbl, src1, dst4, zeros_nw, out, sidx, didx, rows0, rows1, acc,
            sem0, sem1):
        c = lax.axis_index("c")
        s = lax.axis_index("s")
        wid = c * NS + s
        pltpu.sync_copy(zeros_nw.at[pl.ds(s * RPT, RPT)], acc.at[pl.ds(s * RPT, RPT)])
        pltpu.sync_copy(src1.at[pl.ds(wid * (KCH * CH), KCH * CH)], sidx)
        pltpu.sync_copy(dst4.at[wid], didx)
        plsc.subcore_barrier()

        def sl(k):
            return sidx.at[pl.ds(k * CH, CH)]

        pltpu.async_copy(tbl.at[sl(0)], rows0, sem0)

        def body(i, carry):
            k0 = i * 2
            pltpu.async_copy(tbl.at[sl(k0 + 1)], rows1, sem1)
            pltpu.make_async_copy(tbl.at[sl(k0)], rows0, sem0).wait()
            pltpu.sync_copy(rows0, acc.at[didx.at[k0, 0]], add=True)

            @pl.when(k0 + 2 < KCH)
            def _():
                pltpu.async_copy(tbl.at[sl(k0 + 2)], rows0, sem0)

            pltpu.make_async_copy(tbl.at[sl(k0 + 1)], rows1, sem1).wait()
            pltpu.sync_copy(rows1, acc.at[didx.at[k0 + 1, 0]], add=True)
            return carry

        lax.fori_loop(0, KCH // 2, body, 0)
        if KCH % 2:
            pltpu.make_async_copy(tbl.at[sl(KCH - 1)], rows0, sem0).wait()
            pltpu.sync_copy(rows0, acc.at[didx.at[KCH - 1, 0]], add=True)
        plsc.subcore_barrier()
        pltpu.sync_copy(acc.at[pl.ds(s * RPT, RPT)], out.at[c, pl.ds(s * RPT, RPT)])

    return agg


_agg128 = _make_agg(HID)

RB = 400
GRID = N // RB


def _mm_scale(x_ref, w_ref, dinv_ref, o_ref):
    o_ref[...] = jnp.dot(x_ref[...], w_ref[...],
                         preferred_element_type=jnp.float32) * dinv_ref[...]


def _layer2(p_ref, h1s_ref, dinv_ref, b1_ref, o_ref):
    a = (p_ref[0] + p_ref[1] + h1s_ref[...]) * dinv_ref[...] + b1_ref[...]
    o_ref[...] = jnp.maximum(a, 0.0) * dinv_ref[...]


def _combine(q_ref, g_ref, dinv_ref, w2_ref, b2_ref, o_ref):
    a = (q_ref[0] + q_ref[1] + g_ref[...]) * dinv_ref[...]
    o_ref[...] = jnp.dot(a, w2_ref[...],
                         preferred_element_type=jnp.float32) + b2_ref[...]


def kernel(x, edge_index, W1, b1, W2, b2):
    ei = edge_index.astype(jnp.int32)
    src1 = ei[0]
    dst4 = ei[1].reshape(NW, KCH, 1, CH)
    zeros_np = jnp.zeros((NP,), jnp.float32)
    zeros_h = jnp.zeros((NAC, HID), jnp.float32)
    b1_2d = b1.reshape(1, HID)
    b2_2d = b2.reshape(1, CLS)

    dinv_col = _deg_dinv(dst4, zeros_np).reshape(NP, 1)

    h1s = pl.pallas_call(
        _mm_scale,
        grid=(GRID,),
        in_specs=[
            pl.BlockSpec((RB, F), lambda r: (r, 0)),
            pl.BlockSpec((F, HID), lambda r: (0, 0)),
            pl.BlockSpec((RB, 1), lambda r: (r, 0)),
        ],
        out_specs=pl.BlockSpec((RB, HID), lambda r: (r, 0)),
        out_shape=jax.ShapeDtypeStruct((N, HID), jnp.float32),
    )(x, W1, dinv_col)

    p = _agg128(h1s, src1, dst4, zeros_h)

    g = pl.pallas_call(
        _layer2,
        grid=(GRID,),
        in_specs=[
            pl.BlockSpec((NC, RB, HID), lambda r: (0, r, 0)),
            pl.BlockSpec((RB, HID), lambda r: (r, 0)),
            pl.BlockSpec((RB, 1), lambda r: (r, 0)),
            pl.BlockSpec((1, HID), lambda r: (0, 0)),
        ],
        out_specs=pl.BlockSpec((RB, HID), lambda r: (r, 0)),
        out_shape=jax.ShapeDtypeStruct((N, HID), jnp.float32),
    )(p, h1s, dinv_col, b1_2d)

    q = _agg128(g, src1, dst4, zeros_h)

    out = pl.pallas_call(
        _combine,
        grid=(GRID,),
        in_specs=[
            pl.BlockSpec((NC, RB, HID), lambda r: (0, r, 0)),
            pl.BlockSpec((RB, HID), lambda r: (r, 0)),
            pl.BlockSpec((RB, 1), lambda r: (r, 0)),
            pl.BlockSpec((HID, CLS), lambda r: (0, 0)),
            pl.BlockSpec((1, CLS), lambda r: (0, 0)),
        ],
        out_specs=pl.BlockSpec((RB, CLS), lambda r: (r, 0)),
        out_shape=jax.ShapeDtypeStruct((N, CLS), jnp.float32),
    )(q, g, dinv_col, W2, b2_2d)
    return out


# speedup vs baseline: 29.6352x; 1.1183x over previous
import functools

import jax
import jax.numpy as jnp
from jax import lax
from jax.experimental import pallas as pl
from jax.experimental.pallas import tpu as pltpu
from jax.experimental.pallas import tpu_sc as plsc

N = 10000
E = 320000
F = 128
HID = 128
CLS = 40
NP = 10240
NAC = 10112
NC, NS = 2, 16
NW = NC * NS
CH = 80
KCH = E // NW // CH
RPT = NAC // NS
NPW = NP // NW
KDEG = 2 * KCH

_MESH = plsc.VectorSubcoreMesh(core_axis_name="c", subcore_axis_name="s")


@functools.partial(
    pl.kernel,
    out_type=jax.ShapeDtypeStruct((NP,), jnp.float32),
    mesh=_MESH,
    scratch_types=[
        pltpu.VMEM((KDEG, 1, CH), jnp.int32),
        pltpu.VMEM((128,), jnp.float32),
        pltpu.VMEM((NPW,), jnp.float32),
        pltpu.VMEM_SHARED((NP,), jnp.float32),
        pltpu.SemaphoreType.DMA,
    ],
)
def _deg_dinv(dst3, zeros_np, dinv_out, didx, ones_v, dbuf, hist, sem):
    c = lax.axis_index("c")
    s = lax.axis_index("s")
    wid = c * NS + s
    npt = NP // NS
    pltpu.sync_copy(dst3.at[pl.ds(s * KDEG, KDEG)], didx)
    pltpu.sync_copy(zeros_np.at[pl.ds(s * npt, npt)], hist.at[pl.ds(s * npt, npt)])
    for j in range(8):
        ones_v[pl.ds(j * 16, 16)] = jnp.full((16,), 1.0, jnp.float32)
    plsc.subcore_barrier()

    def body(k, carry):
        pltpu.sync_copy(ones_v.at[pl.ds(0, CH)], hist.at[didx.at[k, 0]],
                        add=True)
        return carry

    lax.fori_loop(0, KDEG, body, 0)
    plsc.subcore_barrier()

    pltpu.sync_copy(hist.at[pl.ds(wid * NPW, NPW)], dbuf)
    for i in range(NPW // 16):
        d = dbuf[pl.ds(i * 16, 16)] + 1.0
        seed = jnp.full((16,), 0x5F3759DF, jnp.int32) - lax.shift_right_logical(
            lax.bitcast_convert_type(d, jnp.int32), 1)
        y = lax.bitcast_convert_type(seed, jnp.float32)
        for _ in range(3):
            y = y * (1.5 - 0.5 * d * y * y)
        dbuf[pl.ds(i * 16, 16)] = y
    pltpu.sync_copy(dbuf, dinv_out.at[pl.ds(wid * NPW, NPW)])


def _make_agg(width):
    @functools.partial(
        pl.kernel,
        out_type=jax.ShapeDtypeStruct((NC, NAC, width), jnp.float32),
        mesh=_MESH,
        scratch_types=[
            pltpu.VMEM((4, 1, CH), jnp.int32),
            pltpu.VMEM((4, 1, CH), jnp.int32),
            [pltpu.VMEM((CH, width), jnp.float32) for _ in range(4)],
            pltpu.VMEM_SHARED((NAC, width), jnp.float32),
            [pltpu.SemaphoreType.DMA for _ in range(4)],
            [pltpu.SemaphoreType.DMA for _ in range(4)],
        ],
    )
    def agg(tbl, src3, dst3, zeros_nw, out, sstage, dstage, rows, acc,
            isem, gsem):
        c = lax.axis_index("c")
        s = lax.axis_index("s")
        wid = c * NS + s
        base = wid * KCH

        def fetch(k, j):
            pltpu.async_copy(src3.at[base + k], sstage.at[j], isem[j])
            pltpu.async_copy(dst3.at[base + k], dstage.at[j], isem[j])

        def fetch_wait(k, j):
            pltpu.make_async_copy(src3.at[base + k], sstage.at[j], isem[j]).wait()
            pltpu.make_async_copy(dst3.at[base + k], dstage.at[j], isem[j]).wait()

        def gather(k, j):
            pltpu.async_copy(tbl.at[sstage.at[j, 0]], rows[j], gsem[j])

        def gather_wait(k, j):
            pltpu.make_async_copy(tbl.at[sstage.at[j, 0]], rows[j], gsem[j]).wait()

        for j in range(4):
            fetch(j, j)
        pltpu.sync_copy(zeros_nw.at[pl.ds(s * RPT, RPT)], acc.at[pl.ds(s * RPT, RPT)])
        for j in range(2):
            fetch_wait(j, j)
            gather(j, j)
        plsc.subcore_barrier()

        def body(i, carry):
            for j in range(4):
                k = i * 4 + j
                gather_wait(k, j)
                pltpu.sync_copy(rows[j], acc.at[dstage.at[j, 0]], add=True)

                @pl.when(k + 4 < KCH)
                def _():
                    fetch(k + 4, j)

                @pl.when(k + 2 < KCH)
                def _():
                    j2 = (j + 2) % 4
                    fetch_wait(k + 2, j2)
                    gather(k + 2, j2)
            return carry

        lax.fori_loop(0, KCH // 4, body, 0)
        for k in range(KCH - KCH % 4, KCH):
            j = k % 4
            gather_wait(k, j)
            pltpu.sync_copy(rows[j], acc.at[dstage.at[j, 0]], add=True)
        plsc.subcore_barrier()
        pltpu.sync_copy(acc.at[pl.ds(s * RPT, RPT)], out.at[c, pl.ds(s * RPT, RPT)])

    return agg


_agg128 = _make_agg(HID)

RB = 2000
GRID = N // RB


def _mm_scale(x_ref, w_ref, dinv_ref, o_ref):
    o_ref[...] = jnp.dot(x_ref[...], w_ref[...],
                         preferred_element_type=jnp.float32) * dinv_ref[...]


def _layer2(p_ref, h1s_ref, dinv_ref, b1_ref, o_ref):
    a = (p_ref[0] + p_ref[1] + h1s_ref[...]) * dinv_ref[...] + b1_ref[...]
    o_ref[...] = jnp.maximum(a, 0.0) * dinv_ref[...]


def _combine(q_ref, g_ref, dinv_ref, w2_ref, b2_ref, o_ref):
    a = (q_ref[0] + q_ref[1] + g_ref[...]) * dinv_ref[...]
    o_ref[...] = jnp.dot(a, w2_ref[...],
                         preferred_element_type=jnp.float32) + b2_ref[...]


def kernel(x, edge_index, W1, b1, W2, b2):
    ei = edge_index.astype(jnp.int32)
    src3 = ei[0].reshape(NW * KCH, 1, CH)
    dst3 = ei[1].reshape(NW * KCH, 1, CH)
    zeros_np = jnp.zeros((NP,), jnp.float32)
    zeros_h = jnp.zeros((NAC, HID), jnp.float32)
    b1_2d = b1.reshape(1, HID)
    b2_2d = b2.reshape(1, CLS)

    dinv_col = _deg_dinv(dst3, zeros_np).reshape(NP, 1)

    h1s = pl.pallas_call(
        _mm_scale,
        grid=(GRID,),
        in_specs=[
            pl.BlockSpec((RB, F), lambda r: (r, 0)),
            pl.BlockSpec((F, HID), lambda r: (0, 0)),
            pl.BlockSpec((RB, 1), lambda r: (r, 0)),
        ],
        out_specs=pl.BlockSpec((RB, HID), lambda r: (r, 0)),
        out_shape=jax.ShapeDtypeStruct((N, HID), jnp.float32),
    )(x, W1, dinv_col)

    p = _agg128(h1s, src3, dst3, zeros_h)

    g = pl.pallas_call(
        _layer2,
        grid=(GRID,),
        in_specs=[
            pl.BlockSpec((NC, RB, HID), lambda r: (0, r, 0)),
            pl.BlockSpec((RB, HID), lambda r: (r, 0)),
            pl.BlockSpec((RB, 1), lambda r: (r, 0)),
            pl.BlockSpec((1, HID), lambda r: (0, 0)),
        ],
        out_specs=pl.BlockSpec((RB, HID), lambda r: (r, 0)),
        out_shape=jax.ShapeDtypeStruct((N, HID), jnp.float32),
    )(p, h1s, dinv_col, b1_2d)

    q = _agg128(g, src3, dst3, zeros_h)

    out = pl.pallas_call(
        _combine,
        grid=(GRID,),
        in_specs=[
            pl.BlockSpec((NC, RB, HID), lambda r: (0, r, 0)),
            pl.BlockSpec((RB, HID), lambda r: (r, 0)),
            pl.BlockSpec((RB, 1), lambda r: (r, 0)),
            pl.BlockSpec((HID, CLS), lambda r: (0, 0)),
            pl.BlockSpec((1, CLS), lambda r: (0, 0)),
        ],
        out_specs=pl.BlockSpec((RB, CLS), lambda r: (r, 0)),
        out_shape=jax.ShapeDtypeStruct((N, CLS), jnp.float32),
    )(q, g, dinv_col, W2, b2_2d)
    return out


# speedup vs baseline: 34.2563x; 1.1559x over previous
import functools

import jax
import jax.numpy as jnp
from jax import lax
from jax.experimental import pallas as pl
from jax.experimental.pallas import tpu as pltpu
from jax.experimental.pallas import tpu_sc as plsc

N = 10000
E = 320000
F = 128
HID = 128
CLS = 40
NP = 10240
NAC = 10112
NC, NS = 2, 16
NW = NC * NS
CH = 100
KCH = E // NW // CH
RPT = NAC // NS
NPW = NP // NW
KDEG = 2 * KCH

_MESH = plsc.VectorSubcoreMesh(core_axis_name="c", subcore_axis_name="s")


@functools.partial(
    pl.kernel,
    out_type=jax.ShapeDtypeStruct((NP,), jnp.float32),
    mesh=_MESH,
    scratch_types=[
        pltpu.VMEM((KDEG, 1, CH), jnp.int32),
        pltpu.VMEM((128,), jnp.float32),
        pltpu.VMEM((NPW,), jnp.float32),
        pltpu.VMEM_SHARED((NP,), jnp.float32),
        pltpu.SemaphoreType.DMA,
    ],
)
def _deg_dinv(dst3, zeros_np, dinv_out, didx, ones_v, dbuf, hist, sem):
    c = lax.axis_index("c")
    s = lax.axis_index("s")
    wid = c * NS + s
    npt = NP // NS
    pltpu.sync_copy(dst3.at[pl.ds(s * KDEG, KDEG)], didx)
    pltpu.sync_copy(zeros_np.at[pl.ds(s * npt, npt)], hist.at[pl.ds(s * npt, npt)])
    for j in range(8):
        ones_v[pl.ds(j * 16, 16)] = jnp.full((16,), 1.0, jnp.float32)
    plsc.subcore_barrier()

    def body(k, carry):
        pltpu.sync_copy(ones_v.at[pl.ds(0, CH)], hist.at[didx.at[k, 0]],
                        add=True)
        return carry

    lax.fori_loop(0, KDEG, body, 0)
    plsc.subcore_barrier()

    pltpu.sync_copy(hist.at[pl.ds(wid * NPW, NPW)], dbuf)
    for i in range(NPW // 16):
        d = dbuf[pl.ds(i * 16, 16)] + 1.0
        seed = jnp.full((16,), 0x5F3759DF, jnp.int32) - lax.shift_right_logical(
            lax.bitcast_convert_type(d, jnp.int32), 1)
        y = lax.bitcast_convert_type(seed, jnp.float32)
        for _ in range(3):
            y = y * (1.5 - 0.5 * d * y * y)
        dbuf[pl.ds(i * 16, 16)] = y
    pltpu.sync_copy(dbuf, dinv_out.at[pl.ds(wid * NPW, NPW)])


def _make_agg(width):
    @functools.partial(
        pl.kernel,
        out_type=jax.ShapeDtypeStruct((NC, NAC, width), jnp.float32),
        mesh=_MESH,
        scratch_types=[
            pltpu.VMEM((6, 1, CH), jnp.int32),
            pltpu.VMEM((6, 1, CH), jnp.int32),
            [pltpu.VMEM((CH, width), jnp.float32) for _ in range(3)],
            pltpu.VMEM_SHARED((NAC, width), jnp.float32),
            [pltpu.SemaphoreType.DMA for _ in range(6)],
            [pltpu.SemaphoreType.DMA for _ in range(3)],
            pltpu.SemaphoreType.DMA,
        ],
    )
    def agg(tbl, src3, dst3, zeros_nw, out, sstage, dstage, rows, acc,
            isem, gsem, ssem):
        c = lax.axis_index("c")
        s = lax.axis_index("s")
        wid = c * NS + s
        base = wid * KCH

        def fetch(k, m):
            pltpu.async_copy(src3.at[base + k], sstage.at[m], isem[m])
            pltpu.async_copy(dst3.at[base + k], dstage.at[m], isem[m])

        def fetch_wait(k, m):
            pltpu.make_async_copy(src3.at[base + k], sstage.at[m], isem[m]).wait()
            pltpu.make_async_copy(dst3.at[base + k], dstage.at[m], isem[m]).wait()

        def gather(m, j):
            pltpu.async_copy(tbl.at[sstage.at[m, 0]], rows[j], gsem[j])

        def gather_wait(m, j):
            pltpu.make_async_copy(tbl.at[sstage.at[m, 0]], rows[j], gsem[j]).wait()

        def scatter_wait(m, j):
            pltpu.make_async_copy(rows[j], acc.at[dstage.at[m, 0]], ssem).wait()

        for m in range(4):
            fetch(m, m)
        pltpu.sync_copy(zeros_nw.at[pl.ds(s * RPT, RPT)], acc.at[pl.ds(s * RPT, RPT)])
        for k in range(2):
            fetch_wait(k, k)
            gather(k, k)
        plsc.subcore_barrier()

        def phase(k, j, m):
            k = jnp.int32(k)
            gather_wait(m, j)

            @pl.when(k > 0)
            def _():
                scatter_wait((m + 5) % 6, (j + 2) % 3)

            pltpu.async_copy(rows[j], acc.at[dstage.at[m, 0]], ssem, add=True)

            @pl.when(k + 4 < KCH)
            def _():
                fetch(k + 4, (m + 4) % 6)

            @pl.when(k + 2 < KCH)
            def _():
                fetch_wait(k + 2, (m + 2) % 6)
                gather((m + 2) % 6, (j + 2) % 3)

        def body(i, carry):
            for u in range(6):
                phase(i * 6 + u, u % 3, u)
            return carry

        lax.fori_loop(0, KCH // 6, body, 0)
        for k in range(KCH - KCH % 6, KCH):
            phase(k, k % 3, k % 6)
        scatter_wait((KCH - 1) % 6, (KCH - 1) % 3)
        plsc.subcore_barrier()
        pltpu.sync_copy(acc.at[pl.ds(s * RPT, RPT)], out.at[c, pl.ds(s * RPT, RPT)])

    return agg


_agg128 = _make_agg(HID)

RB = 2000
GRID = N // RB


def _mm_scale(x_ref, w_ref, dinv_ref, o_ref):
    o_ref[...] = jnp.dot(x_ref[...], w_ref[...],
                         preferred_element_type=jnp.float32) * dinv_ref[...]


def _layer2(p_ref, h1s_ref, dinv_ref, b1_ref, o_ref):
    a = (p_ref[0] + p_ref[1] + h1s_ref[...]) * dinv_ref[...] + b1_ref[...]
    o_ref[...] = jnp.maximum(a, 0.0) * dinv_ref[...]


def _combine(q_ref, g_ref, dinv_ref, w2_ref, b2_ref, o_ref):
    a = (q_ref[0] + q_ref[1] + g_ref[...]) * dinv_ref[...]
    o_ref[...] = jnp.dot(a, w2_ref[...],
                         preferred_element_type=jnp.float32) + b2_ref[...]


def kernel(x, edge_index, W1, b1, W2, b2):
    ei = edge_index.astype(jnp.int32)
    src3 = ei[0].reshape(NW * KCH, 1, CH)
    dst3 = ei[1].reshape(NW * KCH, 1, CH)
    zeros_np = jnp.zeros((NP,), jnp.float32)
    zeros_h = jnp.zeros((NAC, HID), jnp.float32)
    b1_2d = b1.reshape(1, HID)
    b2_2d = b2.reshape(1, CLS)

    dinv_col = _deg_dinv(dst3, zeros_np).reshape(NP, 1)

    h1s = pl.pallas_call(
        _mm_scale,
        grid=(GRID,),
        in_specs=[
            pl.BlockSpec((RB, F), lambda r: (r, 0)),
            pl.BlockSpec((F, HID), lambda r: (0, 0)),
            pl.BlockSpec((RB, 1), lambda r: (r, 0)),
        ],
        out_specs=pl.BlockSpec((RB, HID), lambda r: (r, 0)),
        out_shape=jax.ShapeDtypeStruct((N, HID), jnp.float32),
    )(x, W1, dinv_col)

    p = _agg128(h1s, src3, dst3, zeros_h)

    g = pl.pallas_call(
        _layer2,
        grid=(GRID,),
        in_specs=[
            pl.BlockSpec((NC, RB, HID), lambda r: (0, r, 0)),
            pl.BlockSpec((RB, HID), lambda r: (r, 0)),
            pl.BlockSpec((RB, 1), lambda r: (r, 0)),
            pl.BlockSpec((1, HID), lambda r: (0, 0)),
        ],
        out_specs=pl.BlockSpec((RB, HID), lambda r: (r, 0)),
        out_shape=jax.ShapeDtypeStruct((N, HID), jnp.float32),
    )(p, h1s, dinv_col, b1_2d)

    q = _agg128(g, src3, dst3, zeros_h)

    out = pl.pallas_call(
        _combine,
        grid=(GRID,),
        in_specs=[
            pl.BlockSpec((NC, RB, HID), lambda r: (0, r, 0)),
            pl.BlockSpec((RB, HID), lambda r: (r, 0)),
            pl.BlockSpec((RB, 1), lambda r: (r, 0)),
            pl.BlockSpec((HID, CLS), lambda r: (0, 0)),
            pl.BlockSpec((1, CLS), lambda r: (0, 0)),
        ],
        out_specs=pl.BlockSpec((RB, CLS), lambda r: (r, 0)),
        out_shape=jax.ShapeDtypeStruct((N, CLS), jnp.float32),
    )(q, g, dinv_col, W2, b2_2d)
    return out


# speedup vs baseline: 35.4172x; 1.0339x over previous
import functools

import jax
import jax.numpy as jnp
from jax import lax
from jax.experimental import pallas as pl
from jax.experimental.pallas import tpu as pltpu
from jax.experimental.pallas import tpu_sc as plsc

N = 10000
E = 320000
F = 128
HID = 128
CLS = 40
NP = 10240
NAC = 10112
NC, NS = 2, 16
NW = NC * NS
CH = 100
KCH = E // NW // CH
RPT = NAC // NS
NPW = NP // NW
KDEG = 2 * KCH

_MESH = plsc.VectorSubcoreMesh(core_axis_name="c", subcore_axis_name="s")


@functools.partial(
    pl.kernel,
    out_type=jax.ShapeDtypeStruct((NC, NP), jnp.float32),
    mesh=_MESH,
    scratch_types=[
        pltpu.VMEM((KCH, 1, CH), jnp.int32),
        pltpu.VMEM((128,), jnp.float32),
        pltpu.VMEM_SHARED((NP,), jnp.float32),
    ],
)
def _deg_hist(dst3, zeros_np, hist_out, didx, ones_v, hist):
    c = lax.axis_index("c")
    s = lax.axis_index("s")
    wid = c * NS + s
    npt = NP // NS
    pltpu.sync_copy(dst3.at[pl.ds(wid * KCH, KCH)], didx)
    pltpu.sync_copy(zeros_np.at[pl.ds(s * npt, npt)], hist.at[pl.ds(s * npt, npt)])
    for j in range(8):
        ones_v[pl.ds(j * 16, 16)] = jnp.full((16,), 1.0, jnp.float32)
    plsc.subcore_barrier()

    def body(k, carry):
        pltpu.sync_copy(ones_v.at[pl.ds(0, CH)], hist.at[didx.at[k, 0]],
                        add=True)
        return carry

    lax.fori_loop(0, KCH, body, 0)
    plsc.subcore_barrier()
    pltpu.sync_copy(hist.at[pl.ds(s * npt, npt)],
                    hist_out.at[c, pl.ds(s * npt, npt)])


def _dinv_col(h_ref, o_ref):
    deg = 1.0 + h_ref[0] + h_ref[1]
    o_ref[...] = lax.rsqrt(deg).reshape(NP, 1)


def _make_agg(width):
    @functools.partial(
        pl.kernel,
        out_type=jax.ShapeDtypeStruct((NC, NAC, width), jnp.float32),
        mesh=_MESH,
        scratch_types=[
            pltpu.VMEM((6, 1, CH), jnp.int32),
            pltpu.VMEM((6, 1, CH), jnp.int32),
            [pltpu.VMEM((CH, width), jnp.float32) for _ in range(3)],
            pltpu.VMEM_SHARED((NAC, width), jnp.float32),
            [pltpu.SemaphoreType.DMA for _ in range(6)],
            [pltpu.SemaphoreType.DMA for _ in range(3)],
            pltpu.SemaphoreType.DMA,
        ],
    )
    def agg(tbl, src3, dst3, zeros_nw, out, sstage, dstage, rows, acc,
            isem, gsem, ssem):
        c = lax.axis_index("c")
        s = lax.axis_index("s")
        wid = c * NS + s
        base = wid * KCH

        def fetch(k, m):
            pltpu.async_copy(src3.at[base + k], sstage.at[m], isem[m])
            pltpu.async_copy(dst3.at[base + k], dstage.at[m], isem[m])

        def fetch_wait(k, m):
            pltpu.make_async_copy(src3.at[base + k], sstage.at[m], isem[m]).wait()
            pltpu.make_async_copy(dst3.at[base + k], dstage.at[m], isem[m]).wait()

        def gather(m, j):
            pltpu.async_copy(tbl.at[sstage.at[m, 0]], rows[j], gsem[j])

        def gather_wait(m, j):
            pltpu.make_async_copy(tbl.at[sstage.at[m, 0]], rows[j], gsem[j]).wait()

        def scatter_wait(m, j):
            pltpu.make_async_copy(rows[j], acc.at[dstage.at[m, 0]], ssem).wait()

        for m in range(4):
            fetch(m, m)
        pltpu.sync_copy(zeros_nw.at[pl.ds(s * RPT, RPT)], acc.at[pl.ds(s * RPT, RPT)])
        for k in range(2):
            fetch_wait(k, k)
            gather(k, k)
        plsc.subcore_barrier()

        def phase(k, j, m):
            k = jnp.int32(k)
            gather_wait(m, j)

            @pl.when(k > 0)
            def _():
                scatter_wait((m + 5) % 6, (j + 2) % 3)

            pltpu.async_copy(rows[j], acc.at[dstage.at[m, 0]], ssem, add=True)

            @pl.when(k + 4 < KCH)
            def _():
                fetch(k + 4, (m + 4) % 6)

            @pl.when(k + 2 < KCH)
            def _():
                fetch_wait(k + 2, (m + 2) % 6)
                gather((m + 2) % 6, (j + 2) % 3)

        def body(i, carry):
            for u in range(6):
                phase(i * 6 + u, u % 3, u)
            return carry

        lax.fori_loop(0, KCH // 6, body, 0)
        for k in range(KCH - KCH % 6, KCH):
            phase(k, k % 3, k % 6)
        scatter_wait((KCH - 1) % 6, (KCH - 1) % 3)
        plsc.subcore_barrier()
        pltpu.sync_copy(acc.at[pl.ds(s * RPT, RPT)], out.at[c, pl.ds(s * RPT, RPT)])

    return agg


_agg128 = _make_agg(HID)

RB = 2000
GRID = N // RB


def _mm_scale(x_ref, w_ref, dinv_ref, o_ref):
    o_ref[...] = jnp.dot(x_ref[...], w_ref[...],
                         preferred_element_type=jnp.float32) * dinv_ref[...]


def _layer2(p_ref, h1s_ref, dinv_ref, b1_ref, o_ref):
    a = (p_ref[0] + p_ref[1] + h1s_ref[...]) * dinv_ref[...] + b1_ref[...]
    o_ref[...] = jnp.maximum(a, 0.0) * dinv_ref[...]


def _combine(q_ref, g_ref, dinv_ref, w2_ref, b2_ref, o_ref):
    a = (q_ref[0] + q_ref[1] + g_ref[...]) * dinv_ref[...]
    o_ref[...] = jnp.dot(a, w2_ref[...],
                         preferred_element_type=jnp.float32) + b2_ref[...]


def kernel(x, edge_index, W1, b1, W2, b2):
    ei = edge_index.astype(jnp.int32)
    src3 = ei[0].reshape(NW * KCH, 1, CH)
    dst3 = ei[1].reshape(NW * KCH, 1, CH)
    zeros_np = jnp.zeros((NP,), jnp.float32)
    zeros_h = jnp.zeros((NAC, HID), jnp.float32)
    b1_2d = b1.reshape(1, HID)
    b2_2d = b2.reshape(1, CLS)

    hist = _deg_hist(dst3, zeros_np)
    dinv_col = pl.pallas_call(
        _dinv_col,
        in_specs=[pl.BlockSpec((NC, NP), lambda: (0, 0))],
        out_specs=pl.BlockSpec((NP, 1), lambda: (0, 0)),
        out_shape=jax.ShapeDtypeStruct((NP, 1), jnp.float32),
    )(hist)

    h1s = pl.pallas_call(
        _mm_scale,
        grid=(GRID,),
        in_specs=[
            pl.BlockSpec((RB, F), lambda r: (r, 0)),
            pl.BlockSpec((F, HID), lambda r: (0, 0)),
            pl.BlockSpec((RB, 1), lambda r: (r, 0)),
        ],
        out_specs=pl.BlockSpec((RB, HID), lambda r: (r, 0)),
        out_shape=jax.ShapeDtypeStruct((N, HID), jnp.float32),
    )(x, W1, dinv_col)

    p = _agg128(h1s, src3, dst3, zeros_h)

    g = pl.pallas_call(
        _layer2,
        grid=(GRID,),
        in_specs=[
            pl.BlockSpec((NC, RB, HID), lambda r: (0, r, 0)),
            pl.BlockSpec((RB, HID), lambda r: (r, 0)),
            pl.BlockSpec((RB, 1), lambda r: (r, 0)),
            pl.BlockSpec((1, HID), lambda r: (0, 0)),
        ],
        out_specs=pl.BlockSpec((RB, HID), lambda r: (r, 0)),
        out_shape=jax.ShapeDtypeStruct((N, HID), jnp.float32),
    )(p, h1s, dinv_col, b1_2d)

    q = _agg128(g, src3, dst3, zeros_h)

    out = pl.pallas_call(
        _combine,
        grid=(GRID,),
        in_specs=[
            pl.BlockSpec((NC, RB, HID), lambda r: (0, r, 0)),
            pl.BlockSpec((RB, HID), lambda r: (r, 0)),
            pl.BlockSpec((RB, 1), lambda r: (r, 0)),
            pl.BlockSpec((HID, CLS), lambda r: (0, 0)),
            pl.BlockSpec((1, CLS), lambda r: (0, 0)),
        ],
        out_specs=pl.BlockSpec((RB, CLS), lambda r: (r, 0)),
        out_shape=jax.ShapeDtypeStruct((N, CLS), jnp.float32),
    )(q, g, dinv_col, W2, b2_2d)
    return out


# speedup vs baseline: 35.4816x; 1.0018x over previous
import functools

import jax
import jax.numpy as jnp
from jax import lax
from jax.experimental import pallas as pl
from jax.experimental.pallas import tpu as pltpu
from jax.experimental.pallas import tpu_sc as plsc

N = 10000
E = 320000
F = 128
HID = 128
CLS = 40
NP = 10240
NAC = 10112
NC, NS = 2, 16
NW = NC * NS
CH = 100
KCH = E // NW // CH
RPT = NAC // NS
NPW = NP // NW
KDEG = 2 * KCH

_MESH = plsc.VectorSubcoreMesh(core_axis_name="c", subcore_axis_name="s")


@functools.partial(
    pl.kernel,
    out_type=jax.ShapeDtypeStruct((NC, NP), jnp.float32),
    mesh=_MESH,
    scratch_types=[
        pltpu.VMEM((KCH, 1, CH), jnp.int32),
        pltpu.VMEM((128,), jnp.float32),
        pltpu.VMEM_SHARED((NP,), jnp.float32),
    ],
)
def _deg_hist(dst3, zeros_np, hist_out, didx, ones_v, hist):
    c = lax.axis_index("c")
    s = lax.axis_index("s")
    wid = c * NS + s
    npt = NP // NS
    pltpu.sync_copy(dst3.at[pl.ds(wid * KCH, KCH)], didx)
    pltpu.sync_copy(zeros_np.at[pl.ds(s * npt, npt)], hist.at[pl.ds(s * npt, npt)])
    for j in range(8):
        ones_v[pl.ds(j * 16, 16)] = jnp.full((16,), 1.0, jnp.float32)
    plsc.subcore_barrier()

    def body(k, carry):
        pltpu.sync_copy(ones_v.at[pl.ds(0, CH)], hist.at[didx.at[k, 0]],
                        add=True)
        return carry

    lax.fori_loop(0, KCH, body, 0)
    plsc.subcore_barrier()
    pltpu.sync_copy(hist.at[pl.ds(s * npt, npt)],
                    hist_out.at[c, pl.ds(s * npt, npt)])


def _dinv_col(h_ref, o_ref):
    deg = 1.0 + h_ref[0] + h_ref[1]
    o_ref[...] = lax.rsqrt(deg).reshape(NP, 1)


def _make_agg(width):
    @functools.partial(
        pl.kernel,
        out_type=jax.ShapeDtypeStruct((NC, NAC, width), jnp.float32),
        mesh=_MESH,
        scratch_types=[
            pltpu.VMEM((6, 1, CH), jnp.int32),
            pltpu.VMEM((6, 1, CH), jnp.int32),
            [pltpu.VMEM((CH, width), jnp.float32) for _ in range(3)],
            pltpu.VMEM_SHARED((NAC, width), jnp.float32),
            [pltpu.SemaphoreType.DMA for _ in range(6)],
            [pltpu.SemaphoreType.DMA for _ in range(3)],
            pltpu.SemaphoreType.DMA,
        ],
    )
    def agg(tbl, src3, dst3, zeros_nw, out, sstage, dstage, rows, acc,
            isem, gsem, ssem):
        c = lax.axis_index("c")
        s = lax.axis_index("s")
        wid = c * NS + s
        base = wid * KCH

        def fetch(k, m):
            pltpu.async_copy(src3.at[base + k], sstage.at[m], isem[m])
            pltpu.async_copy(dst3.at[base + k], dstage.at[m], isem[m])

        def fetch_wait(k, m):
            pltpu.make_async_copy(src3.at[base + k], sstage.at[m], isem[m]).wait()
            pltpu.make_async_copy(dst3.at[base + k], dstage.at[m], isem[m]).wait()

        def gather(m, j):
            pltpu.async_copy(tbl.at[sstage.at[m, 0]], rows[j], gsem[j])

        def gather_wait(m, j):
            pltpu.make_async_copy(tbl.at[sstage.at[m, 0]], rows[j], gsem[j]).wait()

        def scatter_wait(m, j):
            pltpu.make_async_copy(rows[j], acc.at[dstage.at[m, 0]], ssem).wait()

        for m in range(4):
            fetch(m, m)
        pltpu.sync_copy(zeros_nw.at[pl.ds(s * RPT, RPT)], acc.at[pl.ds(s * RPT, RPT)])
        for k in range(2):
            fetch_wait(k, k)
            gather(k, k)
        plsc.subcore_barrier()

        def phase(k, j, m):
            k = jnp.int32(k)
            gather_wait(m, j)

            @pl.when(k > 0)
            def _():
                scatter_wait((m + 5) % 6, (j + 2) % 3)

            pltpu.async_copy(rows[j], acc.at[dstage.at[m, 0]], ssem, add=True)

            @pl.when(k + 4 < KCH)
            def _():
                fetch(k + 4, (m + 4) % 6)

            @pl.when(k + 2 < KCH)
            def _():
                fetch_wait(k + 2, (m + 2) % 6)
                gather((m + 2) % 6, (j + 2) % 3)

        def body(i, carry):
            for u in range(6):
                phase(i * 6 + u, u % 3, u)
            return carry

        lax.fori_loop(0, KCH // 6, body, 0)
        for k in range(KCH - KCH % 6, KCH):
            phase(k, k % 3, k % 6)
        scatter_wait((KCH - 1) % 6, (KCH - 1) % 3)
        plsc.subcore_barrier()
        pltpu.sync_copy(acc.at[pl.ds(s * RPT, RPT)], out.at[c, pl.ds(s * RPT, RPT)])

    return agg


_agg128 = _make_agg(HID)

RB = 2000
GRID = N // RB


def _mm(x_ref, w_ref, o_ref):
    o_ref[...] = jnp.dot(x_ref[...], w_ref[...],
                         preferred_element_type=jnp.float32)


def _scale(h_ref, dinv_ref, o_ref):
    o_ref[...] = h_ref[...] * dinv_ref[...]


def _layer2(p_ref, h1s_ref, dinv_ref, b1_ref, o_ref):
    a = (p_ref[0] + p_ref[1] + h1s_ref[...]) * dinv_ref[...] + b1_ref[...]
    o_ref[...] = jnp.maximum(a, 0.0) * dinv_ref[...]


def _combine(q_ref, g_ref, dinv_ref, w2_ref, b2_ref, o_ref):
    a = (q_ref[0] + q_ref[1] + g_ref[...]) * dinv_ref[...]
    o_ref[...] = jnp.dot(a, w2_ref[...],
                         preferred_element_type=jnp.float32) + b2_ref[...]


def kernel(x, edge_index, W1, b1, W2, b2):
    ei = edge_index.astype(jnp.int32)
    src3 = ei[0].reshape(NW * KCH, 1, CH)
    dst3 = ei[1].reshape(NW * KCH, 1, CH)
    zeros_np = jnp.zeros((NP,), jnp.float32)
    zeros_h = jnp.zeros((NAC, HID), jnp.float32)
    b1_2d = b1.reshape(1, HID)
    b2_2d = b2.reshape(1, CLS)

    h1 = pl.pallas_call(
        _mm,
        grid=(GRID,),
        in_specs=[
            pl.BlockSpec((RB, F), lambda r: (r, 0)),
            pl.BlockSpec((F, HID), lambda r: (0, 0)),
        ],
        out_specs=pl.BlockSpec((RB, HID), lambda r: (r, 0)),
        out_shape=jax.ShapeDtypeStruct((N, HID), jnp.float32),
    )(x, W1)

    hist = _deg_hist(dst3, zeros_np)
    dinv_col = pl.pallas_call(
        _dinv_col,
        in_specs=[pl.BlockSpec((NC, NP), lambda: (0, 0))],
        out_specs=pl.BlockSpec((NP, 1), lambda: (0, 0)),
        out_shape=jax.ShapeDtypeStruct((NP, 1), jnp.float32),
    )(hist)

    h1s = pl.pallas_call(
        _scale,
        grid=(GRID,),
        in_specs=[
            pl.BlockSpec((RB, HID), lambda r: (r, 0)),
            pl.BlockSpec((RB, 1), lambda r: (r, 0)),
        ],
        out_specs=pl.BlockSpec((RB, HID), lambda r: (r, 0)),
        out_shape=jax.ShapeDtypeStruct((N, HID), jnp.float32),
    )(h1, dinv_col)

    p = _agg128(h1s, src3, dst3, zeros_h)

    g = pl.pallas_call(
        _layer2,
        grid=(GRID,),
        in_specs=[
            pl.BlockSpec((NC, RB, HID), lambda r: (0, r, 0)),
            pl.BlockSpec((RB, HID), lambda r: (r, 0)),
            pl.BlockSpec((RB, 1), lambda r: (r, 0)),
            pl.BlockSpec((1, HID), lambda r: (0, 0)),
        ],
        out_specs=pl.BlockSpec((RB, HID), lambda r: (r, 0)),
        out_shape=jax.ShapeDtypeStruct((N, HID), jnp.float32),
    )(p, h1s, dinv_col, b1_2d)

    q = _agg128(g, src3, dst3, zeros_h)

    out = pl.pallas_call(
        _combine,
        grid=(GRID,),
        in_specs=[
            pl.BlockSpec((NC, RB, HID), lambda r: (0, r, 0)),
            pl.BlockSpec((RB, HID), lambda r: (r, 0)),
            pl.BlockSpec((RB, 1), lambda r: (r, 0)),
            pl.BlockSpec((HID, CLS), lambda r: (0, 0)),
            pl.BlockSpec((1, CLS), lambda r: (0, 0)),
        ],
        out_specs=pl.BlockSpec((RB, CLS), lambda r: (r, 0)),
        out_shape=jax.ShapeDtypeStruct((N, CLS), jnp.float32),
    )(q, g, dinv_col, W2, b2_2d)
    return out


# speedup vs baseline: 35.6678x; 1.0052x over previous
import functools

import jax
import jax.numpy as jnp
from jax import lax
from jax.experimental import pallas as pl
from jax.experimental.pallas import tpu as pltpu
from jax.experimental.pallas import tpu_sc as plsc

N = 10000
E = 320000
F = 128
HID = 128
CLS = 40
NP = 10240
NAC = 10112
NC, NS = 2, 16
NW = NC * NS
CH = 100
KCH = E // NW // CH
RPT = NAC // NS
NPW = NP // NW
KDEG = 2 * KCH

_MESH = plsc.VectorSubcoreMesh(core_axis_name="c", subcore_axis_name="s")


@functools.partial(
    pl.kernel,
    out_type=jax.ShapeDtypeStruct((NC, NP), jnp.float32),
    mesh=_MESH,
    scratch_types=[
        pltpu.VMEM((KCH, 1, CH), jnp.int32),
        pltpu.VMEM((128,), jnp.float32),
        pltpu.VMEM_SHARED((NP,), jnp.float32),
    ],
)
def _deg_hist(dst3, zeros_np, hist_out, didx, ones_v, hist):
    c = lax.axis_index("c")
    s = lax.axis_index("s")
    wid = c * NS + s
    npt = NP // NS
    pltpu.sync_copy(dst3.at[pl.ds(wid * KCH, KCH)], didx)
    pltpu.sync_copy(zeros_np.at[pl.ds(s * npt, npt)], hist.at[pl.ds(s * npt, npt)])
    for j in range(8):
        ones_v[pl.ds(j * 16, 16)] = jnp.full((16,), 1.0, jnp.float32)
    plsc.subcore_barrier()

    def body(k, carry):
        pltpu.sync_copy(ones_v.at[pl.ds(0, CH)], hist.at[didx.at[k, 0]],
                        add=True)
        return carry

    lax.fori_loop(0, KCH, body, 0)
    plsc.subcore_barrier()
    pltpu.sync_copy(hist.at[pl.ds(s * npt, npt)],
                    hist_out.at[c, pl.ds(s * npt, npt)])


def _dinv_col(h_ref, o_ref):
    deg = 1.0 + h_ref[0] + h_ref[1]
    o_ref[...] = lax.rsqrt(deg).reshape(NP, 1)


def _make_agg(width):
    @functools.partial(
        pl.kernel,
        out_type=jax.ShapeDtypeStruct((NC, NAC, width), jnp.float32),
        mesh=_MESH,
        scratch_types=[
            pltpu.VMEM((6, 1, CH), jnp.int32),
            pltpu.VMEM((6, 1, CH), jnp.int32),
            [pltpu.VMEM((CH, width), jnp.float32) for _ in range(3)],
            pltpu.VMEM_SHARED((NAC, width), jnp.float32),
            [pltpu.SemaphoreType.DMA for _ in range(6)],
            [pltpu.SemaphoreType.DMA for _ in range(3)],
            pltpu.SemaphoreType.DMA,
        ],
    )
    def agg(tbl, src3, dst3, zeros_nw, out, sstage, dstage, rows, acc,
            isem, gsem, ssem):
        c = lax.axis_index("c")
        s = lax.axis_index("s")
        wid = c * NS + s
        base = wid * KCH

        def fetch(k, m):
            pltpu.async_copy(src3.at[base + k], sstage.at[m], isem[m])
            pltpu.async_copy(dst3.at[base + k], dstage.at[m], isem[m])

        def fetch_wait(k, m):
            pltpu.make_async_copy(src3.at[base + k], sstage.at[m], isem[m]).wait()
            pltpu.make_async_copy(dst3.at[base + k], dstage.at[m], isem[m]).wait()

        def gather(m, j):
            pltpu.async_copy(tbl.at[sstage.at[m, 0]], rows[j], gsem[j])

        def gather_wait(m, j):
            pltpu.make_async_copy(tbl.at[sstage.at[m, 0]], rows[j], gsem[j]).wait()

        def scatter_wait(m, j):
            pltpu.make_async_copy(rows[j], acc.at[dstage.at[m, 0]], ssem).wait()

        for m in range(4):
            fetch(m, m)
        for k in range(2):
            fetch_wait(k, k)
            gather(k, k)
        pltpu.sync_copy(zeros_nw.at[pl.ds(s * RPT, RPT)], acc.at[pl.ds(s * RPT, RPT)])
        plsc.subcore_barrier()

        def phase(k, j, m):
            k = jnp.int32(k)
            gather_wait(m, j)

            @pl.when(k > 0)
            def _():
                scatter_wait((m + 5) % 6, (j + 2) % 3)

            pltpu.async_copy(rows[j], acc.at[dstage.at[m, 0]], ssem, add=True)

            @pl.when(k + 4 < KCH)
            def _():
                fetch(k + 4, (m + 4) % 6)

            @pl.when(k + 2 < KCH)
            def _():
                fetch_wait(k + 2, (m + 2) % 6)
                gather((m + 2) % 6, (j + 2) % 3)

        def body(i, carry):
            for u in range(6):
                phase(i * 6 + u, u % 3, u)
            return carry

        lax.fori_loop(0, KCH // 6, body, 0)
        for k in range(KCH - KCH % 6, KCH):
            phase(k, k % 3, k % 6)
        scatter_wait((KCH - 1) % 6, (KCH - 1) % 3)
        plsc.subcore_barrier()
        pltpu.sync_copy(acc.at[pl.ds(s * RPT, RPT)], out.at[c, pl.ds(s * RPT, RPT)])

    return agg


_agg128 = _make_agg(HID)

RB = 2000
GRID = N // RB


def _mm(x_ref, w_ref, o_ref):
    o_ref[...] = jnp.dot(x_ref[...], w_ref[...],
                         preferred_element_type=jnp.float32)


def _scale(h_ref, dinv_ref, o_ref):
    o_ref[...] = h_ref[...] * dinv_ref[...]


def _layer2(p_ref, h1s_ref, dinv_ref, b1_ref, o_ref):
    a = (p_ref[0] + p_ref[1] + h1s_ref[...]) * dinv_ref[...] + b1_ref[...]
    o_ref[...] = jnp.maximum(a, 0.0) * dinv_ref[...]


def _combine(q_ref, g_ref, dinv_ref, w2_ref, b2_ref, o_ref):
    a = (q_ref[0] + q_ref[1] + g_ref[...]) * dinv_ref[...]
    o_ref[...] = jnp.dot(a, w2_ref[...],
                         preferred_element_type=jnp.float32) + b2_ref[...]


def kernel(x, edge_index, W1, b1, W2, b2):
    ei = edge_index.astype(jnp.int32)
    src3 = ei[0].reshape(NW * KCH, 1, CH)
    dst3 = ei[1].reshape(NW * KCH, 1, CH)
    zeros_np = jnp.zeros((NP,), jnp.float32)
    zeros_h = jnp.zeros((NAC, HID), jnp.float32)
    b1_2d = b1.reshape(1, HID)
    b2_2d = b2.reshape(1, CLS)

    h1 = pl.pallas_call(
        _mm,
        grid=(GRID,),
        in_specs=[
            pl.BlockSpec((RB, F), lambda r: (r, 0)),
            pl.BlockSpec((F, HID), lambda r: (0, 0)),
        ],
        out_specs=pl.BlockSpec((RB, HID), lambda r: (r, 0)),
        out_shape=jax.ShapeDtypeStruct((N, HID), jnp.float32),
    )(x, W1)

    hist = _deg_hist(dst3, zeros_np)
    dinv_col = pl.pallas_call(
        _dinv_col,
        in_specs=[pl.BlockSpec((NC, NP), lambda: (0, 0))],
        out_specs=pl.BlockSpec((NP, 1), lambda: (0, 0)),
        out_shape=jax.ShapeDtypeStruct((NP, 1), jnp.float32),
    )(hist)

    h1s = pl.pallas_call(
        _scale,
        grid=(GRID,),
        in_specs=[
            pl.BlockSpec((RB, HID), lambda r: (r, 0)),
            pl.BlockSpec((RB, 1), lambda r: (r, 0)),
        ],
        out_specs=pl.BlockSpec((RB, HID), lambda r: (r, 0)),
        out_shape=jax.ShapeDtypeStruct((N, HID), jnp.float32),
    )(h1, dinv_col)

    p = _agg128(h1s, src3, dst3, zeros_h)

    g = pl.pallas_call(
        _layer2,
        grid=(GRID,),
        in_specs=[
            pl.BlockSpec((NC, RB, HID), lambda r: (0, r, 0)),
            pl.BlockSpec((RB, HID), lambda r: (r, 0)),
            pl.BlockSpec((RB, 1), lambda r: (r, 0)),
            pl.BlockSpec((1, HID), lambda r: (0, 0)),
        ],
        out_specs=pl.BlockSpec((RB, HID), lambda r: (r, 0)),
        out_shape=jax.ShapeDtypeStruct((N, HID), jnp.float32),
    )(p, h1s, dinv_col, b1_2d)

    q = _agg128(g, src3, dst3, zeros_h)

    out = pl.pallas_call(
        _combine,
        grid=(GRID,),
        in_specs=[
            pl.BlockSpec((NC, RB, HID), lambda r: (0, r, 0)),
            pl.BlockSpec((RB, HID), lambda r: (r, 0)),
            pl.BlockSpec((RB, 1), lambda r: (r, 0)),
            pl.BlockSpec((HID, CLS), lambda r: (0, 0)),
            pl.BlockSpec((1, CLS), lambda r: (0, 0)),
        ],
        out_specs=pl.BlockSpec((RB, CLS), lambda r: (r, 0)),
        out_shape=jax.ShapeDtypeStruct((N, CLS), jnp.float32),
    )(q, g, dinv_col, W2, b2_2d)
    return out
